# baseline (speedup 1.0000x reference)
import functools

import jax
import jax.numpy as jnp
import numpy as np
from jax import lax
from jax.experimental import pallas as pl
from jax.experimental.pallas import tpu as pltpu
from jax.experimental.pallas import tpu_sc as plsc

B = 16384
VOCAB = 1000000
EDIM = 16
NB = 7
NF = 2
NSLOT = NF * NB
D = NSLOT * EDIM

NC = 2
NS = 16
NW = NC * NS

ROWS_PER_W = D // NW
GCH = 2048
PAGE_PAD = 104
SMALL_WIN = 400
SMALL = EDIM * 400 + EDIM * PAGE_PAD + 384


def _sc_gather_body(ids_t, pp_idx,
                    tab_flat, small_flat,
                    embt_out, ppt_out,
                    idx_v, col0, col1, g0, g1, s0, s1):
  wid = lax.axis_index("s") * NC + lax.axis_index("c")
  cols = (col0, col1)
  gsems = (g0, g1)
  ssems = (s0, s1)
  store = [None, None]

  for k in range(ROWS_PER_W):
    p = k % 2
    r = wid * ROWS_PER_W + k
    s = r // EDIM
    e = r % EDIM
    pltpu.sync_copy(ids_t.at[pl.ds(s * B, B)], idx_v)
    if store[p] is not None:
      store[p].wait()
    handles = []
    for j in range(B // GCH):
      handles.append(pltpu.async_copy(
          tab_flat.at[pl.ds(e * VOCAB, VOCAB)].at[idx_v.at[pl.ds(j * GCH, GCH)]],
          cols[p].at[pl.ds(j * GCH, GCH)], gsems[p]))
    for h in handles:
      h.wait()
    store[p] = pltpu.async_copy(
        cols[p], embt_out.at[pl.ds(r * B, B)], ssems[p])

  is_page = wid // EDIM
  e2 = wid % EDIM
  col_base = jnp.where(is_page == 0, e2 * 400, EDIM * 400 + e2 * PAGE_PAD)
  pltpu.sync_copy(pp_idx.at[pl.ds(is_page * B, B)], idx_v)
  p = ROWS_PER_W % 2
  if store[p] is not None:
    store[p].wait()
  handles = []
  for j in range(B // GCH):
    handles.append(pltpu.async_copy(
        small_flat.at[pl.ds(col_base, SMALL_WIN)].at[idx_v.at[pl.ds(j * GCH, GCH)]],
        cols[p].at[pl.ds(j * GCH, GCH)], gsems[p]))
  for h in handles:
    h.wait()
  store[p] = pltpu.async_copy(
      cols[p], ppt_out.at[pl.ds(wid * B, B)], ssems[p])
  for h in store:
    if h is not None:
      h.wait()


@functools.lru_cache(maxsize=None)
def _make_sc_gather():
  return pl.kernel(
      _sc_gather_body,
      out_type=(
          jax.ShapeDtypeStruct((D * B,), jnp.float32),
          jax.ShapeDtypeStruct((2 * EDIM * B,), jnp.float32),
      ),
      mesh=plsc.VectorSubcoreMesh(core_axis_name="c", subcore_axis_name="s"),
      compiler_params=pltpu.CompilerParams(use_tc_tiling_on_sc=False),
      scratch_types=[
          pltpu.VMEM((B,), jnp.int32),
          pltpu.VMEM((B,), jnp.float32),
          pltpu.VMEM((B,), jnp.float32),
          pltpu.SemaphoreType.DMA,
          pltpu.SemaphoreType.DMA,
          pltpu.SemaphoreType.DMA,
          pltpu.SemaphoreType.DMA,
      ],
  )


@functools.lru_cache(maxsize=None)
def _selection_mats():
  i = np.arange(D)
  s = i // EDIM
  e = i % EDIM
  f = s // NB
  qsel_t = np.zeros((D, D), np.float32)
  qsel_t[i, (f * NB + 3) * EDIM + e] = 1.0
  seg_t = np.zeros((NSLOT, D), np.float32)
  seg_t[s, i] = 1.0
  rsel_t = seg_t.T.copy()
  asel = np.zeros((D, NF * EDIM), np.float32)
  asel[i, f * EDIM + e] = 1.0
  return (jnp.asarray(qsel_t), jnp.asarray(seg_t),
          jnp.asarray(rsel_t), jnp.asarray(asel))


def _tc_body(xt_ref, ppt_ref, qsel_ref, seg_ref, rsel_ref,
             w1aw_ref, w1wx_ref, w1pp_ref, b1_ref, g1_ref,
             be1_ref, w2_ref, b2_ref, g2_ref, be2_ref, out_ref):
  xt = xt_ref[...]
  qqt = jax.lax.dot(qsel_ref[...], xt, preferred_element_type=jnp.float32)
  prod = xt * qqt
  scores = jax.lax.dot(seg_ref[...], prod,
                       preferred_element_type=jnp.float32) * (1.0 / 4.0)

  def softmax7(sc):
    m = jnp.max(sc, axis=0, keepdims=True)
    ex = jnp.exp(sc - m)
    return ex / jnp.sum(ex, axis=0, keepdims=True)

  aw = jnp.concatenate(
      [softmax7(scores[:NB]), softmax7(scores[NB:2 * NB])], axis=0)

  w_rep = jax.lax.dot(rsel_ref[...], aw, preferred_element_type=jnp.float32)
  wx = xt * w_rep
  h = (jax.lax.dot(w1aw_ref[...], aw, preferred_element_type=jnp.float32)
       + jax.lax.dot(w1wx_ref[...], wx, preferred_element_type=jnp.float32)
       + jax.lax.dot(w1pp_ref[...], ppt_ref[...],
                     preferred_element_type=jnp.float32)
       + b1_ref[...])
  mu = jnp.mean(h, axis=0, keepdims=True)
  var = jnp.mean((h - mu) ** 2, axis=0, keepdims=True)
  h = g1_ref[...] * (h - mu) / jnp.sqrt(var + 1e-3) + be1_ref[...]
  h = jnp.maximum(h, 0.0)

  h2 = jnp.sum(h * w2_ref[...], axis=0, keepdims=True) + b2_ref[...]
  mu2 = jnp.mean(h2, axis=0, keepdims=True)
  var2 = jnp.mean((h2 - mu2) ** 2, axis=0, keepdims=True)
  h2 = g2_ref[...] * (h2 - mu2) / jnp.sqrt(var2 + 1e-3) + be2_ref[...]
  out_ref[...] = jnp.maximum(h2, 0.0)


def kernel(position, page, near_expo_seq_cate2, near_expo_seq_cate3,
           neighbourhood_table, position_table, page_table,
           W1, b1, g1, be1, W2, b2, g2, be2):
  ids_t = jnp.concatenate(
      [near_expo_seq_cate2.T, near_expo_seq_cate3.T], axis=0
  ).reshape(-1).astype(jnp.int32)
  pp_idx = jnp.concatenate(
      [position.astype(jnp.int32), page.astype(jnp.int32)])

  tab_flat = neighbourhood_table.T.reshape(-1)
  small_flat = jnp.concatenate([
      position_table.T.reshape(-1),
      jnp.pad(page_table.T, ((0, 0), (0, PAGE_PAD - 100))).reshape(-1),
      jnp.zeros((384,), jnp.float32)])

  embt_flat, ppt_flat = _make_sc_gather()(ids_t, pp_idx, tab_flat, small_flat)
  embt = embt_flat.reshape(D, B)
  ppt = ppt_flat.reshape(2 * EDIM, B)

  qsel_t, seg_t, rsel_t, asel = _selection_mats()
  w1aw = W1[:NSLOT].T
  w1wx = (asel @ W1[NSLOT:NSLOT + NF * EDIM]).T
  w1pp = W1[NSLOT + NF * EDIM:].T

  blk = 2048
  grid = B // blk
  full = lambda i: (0, 0)
  colb = lambda i: (0, i)
  out = pl.pallas_call(
      _tc_body,
      grid=(grid,),
      in_specs=[
          pl.BlockSpec((D, blk), colb),
          pl.BlockSpec((2 * EDIM, blk), colb),
          pl.BlockSpec((D, D), full),
          pl.BlockSpec((NSLOT, D), full),
          pl.BlockSpec((D, NSLOT), full),
          pl.BlockSpec((8, NSLOT), full),
          pl.BlockSpec((8, D), full),
          pl.BlockSpec((8, 2 * EDIM), full),
          pl.BlockSpec((8, 1), full),
          pl.BlockSpec((8, 1), full),
          pl.BlockSpec((8, 1), full),
          pl.BlockSpec((8, 1), full),
          pl.BlockSpec((1, 1), full),
          pl.BlockSpec((1, 1), full),
          pl.BlockSpec((1, 1), full),
      ],
      out_specs=pl.BlockSpec((1, blk), colb),
      out_shape=jax.ShapeDtypeStruct((1, B), jnp.float32),
  )(embt, ppt, qsel_t, seg_t, rsel_t, w1aw, w1wx, w1pp,
    b1.reshape(8, 1), g1.reshape(8, 1), be1.reshape(8, 1),
    W2.reshape(8, 1), b2.reshape(1, 1), g2.reshape(1, 1), be2.reshape(1, 1))
  return out.reshape(B, 1)


# speedup vs baseline: 3.4793x; 3.4793x over previous
import functools

import jax
import jax.numpy as jnp
import numpy as np
from jax import lax
from jax.experimental import pallas as pl
from jax.experimental.pallas import tpu as pltpu
from jax.experimental.pallas import tpu_sc as plsc

B = 16384
VOCAB = 1000000
EDIM = 16
NB = 7
NF = 2
NSLOT = NF * NB
D = NSLOT * EDIM

NC = 2
NS = 16
NW = NC * NS

CHUNK = 512
GROUP = 2048
EMB_PER_W = B * NSLOT // NW
POS_PER_W = B // NW

TBLK = 8192
NTB = VOCAB // TBLK
TAIL = VOCAB - NTB * TBLK


def _pack8(y):
  y3 = y.reshape(y.shape[0] // 8, 8, EDIM)
  return jnp.concatenate([y3[:, a, :] for a in range(8)], axis=1)


def _transpose_body(tt_ref, out_ref):
  x = tt_ref[...]
  out_ref[...] = _pack8(jnp.transpose(x))


def _sc_gather_body(emb_idx, pos_idx, page_idx,
                    emb_tab, pos_tab, page_tab,
                    emb_out, pos_out, page_out,
                    idx_v, pidx_v, buf0, buf1, g0, g1, s0, s1):
  wid = lax.axis_index("s") * NC + lax.axis_index("c")
  base = wid * EMB_PER_W
  pltpu.sync_copy(emb_idx.at[pl.ds(base, EMB_PER_W)], idx_v)

  bufs = (buf0, buf1)
  gsems = (g0, g1)
  ssems = (s0, s1)
  ngroups = EMB_PER_W // GROUP
  rem = EMB_PER_W - ngroups * GROUP
  store_handles = [None, None]
  for g in range(ngroups + 1):
    width = GROUP if g < ngroups else rem
    p = g % 2
    if store_handles[p] is not None:
      store_handles[p].wait()
    handles = []
    for j in range(width // CHUNK):
      off = g * GROUP + j * CHUNK
      handles.append(pltpu.async_copy(
          emb_tab.at[idx_v.at[pl.ds(off, CHUNK)]],
          bufs[p].at[pl.ds(j * CHUNK, CHUNK)], gsems[p]))
    for h in handles:
      h.wait()
    store_handles[p] = pltpu.async_copy(
        bufs[p].at[pl.ds(0, width)],
        emb_out.at[pl.ds(base + g * GROUP, width)], ssems[p])
  for h in store_handles:
    if h is not None:
      h.wait()

  pbase = wid * POS_PER_W
  for src_idx, tab, out, buf, gsem, ssem in (
      (pos_idx, pos_tab, pos_out, buf0, g0, s0),
      (page_idx, page_tab, page_out, buf1, g1, s1),
  ):
    pltpu.sync_copy(src_idx.at[pl.ds(pbase, POS_PER_W)], pidx_v)
    pltpu.async_copy(
        tab.at[pidx_v], buf.at[pl.ds(0, POS_PER_W)], gsem).wait()
    pltpu.async_copy(
        buf.at[pl.ds(0, POS_PER_W)], out.at[pl.ds(pbase, POS_PER_W)],
        ssem).wait()


@functools.lru_cache(maxsize=None)
def _make_sc_gather():
  return pl.kernel(
      _sc_gather_body,
      out_type=(
          jax.ShapeDtypeStruct((B * NSLOT, EDIM), jnp.float32),
          jax.ShapeDtypeStruct((B, EDIM), jnp.float32),
          jax.ShapeDtypeStruct((B, EDIM), jnp.float32),
      ),
      mesh=plsc.VectorSubcoreMesh(core_axis_name="c", subcore_axis_name="s"),
      compiler_params=pltpu.CompilerParams(use_tc_tiling_on_sc=False),
      scratch_types=[
          pltpu.VMEM((EMB_PER_W,), jnp.int32),
          pltpu.VMEM((POS_PER_W,), jnp.int32),
          pltpu.VMEM((GROUP, EDIM), jnp.float32),
          pltpu.VMEM((GROUP, EDIM), jnp.float32),
          pltpu.SemaphoreType.DMA,
          pltpu.SemaphoreType.DMA,
          pltpu.SemaphoreType.DMA,
          pltpu.SemaphoreType.DMA,
      ],
  )


@functools.lru_cache(maxsize=None)
def _selection_mats():
  i = np.arange(D)
  s = i // EDIM
  e = i % EDIM
  f = s // NB
  qsel = np.zeros((D, D), np.float32)
  qsel[(f * NB + 3) * EDIM + e, i] = 1.0
  seg = np.zeros((D, NSLOT), np.float32)
  seg[i, s] = 1.0
  rsel = seg.T.copy()
  asel = np.zeros((D, NF * EDIM), np.float32)
  asel[i, f * EDIM + e] = 1.0
  return (jnp.asarray(qsel), jnp.asarray(seg),
          jnp.asarray(rsel), jnp.asarray(asel))


def _tc_body(emb_ref, pos_ref, page_ref, qsel_ref, seg_ref, rsel_ref,
             w1aw_ref, w1wx_ref, w1pos_ref, w1page_ref, b1_ref, g1_ref,
             be1_ref, w2_ref, b2_ref, g2_ref, be2_ref, out_ref):
  x = emb_ref[...]
  qq = jax.lax.dot(x, qsel_ref[...], preferred_element_type=jnp.float32)
  prod = x * qq
  scores = jax.lax.dot(prod, seg_ref[...],
                       preferred_element_type=jnp.float32) * (1.0 / 4.0)

  def softmax7(sc):
    m = jnp.max(sc, axis=-1, keepdims=True)
    ex = jnp.exp(sc - m)
    return ex / jnp.sum(ex, axis=-1, keepdims=True)

  aw = jnp.concatenate(
      [softmax7(scores[:, :NB]), softmax7(scores[:, NB:2 * NB])], axis=1)

  w_rep = jax.lax.dot(aw, rsel_ref[...], preferred_element_type=jnp.float32)
  wx = x * w_rep
  h = (jax.lax.dot(aw, w1aw_ref[...], preferred_element_type=jnp.float32)
       + jax.lax.dot(wx, w1wx_ref[...], preferred_element_type=jnp.float32)
       + jax.lax.dot(pos_ref[...], w1pos_ref[...],
                     preferred_element_type=jnp.float32)
       + jax.lax.dot(page_ref[...], w1page_ref[...],
                     preferred_element_type=jnp.float32)
       + b1_ref[...])
  mu = jnp.mean(h, axis=-1, keepdims=True)
  var = jnp.mean((h - mu) ** 2, axis=-1, keepdims=True)
  h = g1_ref[...] * (h - mu) / jnp.sqrt(var + 1e-3) + be1_ref[...]
  h = jnp.maximum(h, 0.0)

  h2 = jnp.sum(h * w2_ref[...], axis=-1, keepdims=True) + b2_ref[...]
  mu2 = jnp.mean(h2, axis=-1, keepdims=True)
  var2 = jnp.mean((h2 - mu2) ** 2, axis=-1, keepdims=True)
  h2 = g2_ref[...] * (h2 - mu2) / jnp.sqrt(var2 + 1e-3) + be2_ref[...]
  out_ref[...] = jnp.maximum(h2, 0.0)


def _pack_table(neighbourhood_table):
  tt = neighbourhood_table.T
  packed = pl.pallas_call(
      _transpose_body,
      grid=(NTB,),
      in_specs=[pl.BlockSpec((EDIM, TBLK), lambda i: (0, i))],
      out_specs=pl.BlockSpec((TBLK // 8, 128), lambda i: (i, 0)),
      out_shape=jax.ShapeDtypeStruct((VOCAB // 8, 128), jnp.float32),
  )(tt)
  tail_packed = neighbourhood_table[NTB * TBLK:].reshape(TAIL // 8, 128)
  packed = lax.dynamic_update_slice(packed, tail_packed, (NTB * TBLK // 8, 0))
  return packed.reshape(VOCAB, EDIM)


def kernel(position, page, near_expo_seq_cate2, near_expo_seq_cate3,
           neighbourhood_table, position_table, page_table,
           W1, b1, g1, be1, W2, b2, g2, be2):
  ids = jnp.concatenate(
      [near_expo_seq_cate2, near_expo_seq_cate3], axis=1
  ).reshape(-1).astype(jnp.int32)
  pos_idx = position.astype(jnp.int32)
  page_idx = page.astype(jnp.int32)

  rowtab = _pack_table(neighbourhood_table)

  emb_rows, pos_rows, page_rows = _make_sc_gather()(
      ids, pos_idx, page_idx, rowtab, position_table, page_table)
  emb_flat = emb_rows.reshape(B, NSLOT * EDIM)

  qsel, seg, rsel, asel = _selection_mats()
  w1aw = W1[:NSLOT]
  w1wx = asel @ W1[NSLOT:NSLOT + NF * EDIM]
  w1pos = W1[NSLOT + NF * EDIM:NSLOT + NF * EDIM + EDIM]
  w1page = W1[NSLOT + NF * EDIM + EDIM:]

  blk = 2048
  grid = B // blk
  full = lambda i: (0, 0)
  row = lambda i: (i, 0)
  out = pl.pallas_call(
      _tc_body,
      grid=(grid,),
      in_specs=[
          pl.BlockSpec((blk, NSLOT * EDIM), row),
          pl.BlockSpec((blk, EDIM), row),
          pl.BlockSpec((blk, EDIM), row),
          pl.BlockSpec(qsel.shape, full),
          pl.BlockSpec(seg.shape, full),
          pl.BlockSpec(rsel.shape, full),
          pl.BlockSpec((NSLOT, 8), full),
          pl.BlockSpec((D, 8), full),
          pl.BlockSpec((EDIM, 8), full),
          pl.BlockSpec((EDIM, 8), full),
          pl.BlockSpec((1, 8), full),
          pl.BlockSpec((1, 8), full),
          pl.BlockSpec((1, 8), full),
          pl.BlockSpec((1, 8), full),
          pl.BlockSpec((1, 1), full),
          pl.BlockSpec((1, 1), full),
          pl.BlockSpec((1, 1), full),
      ],
      out_specs=pl.BlockSpec((blk, 1), row),
      out_shape=jax.ShapeDtypeStruct((B, 1), jnp.float32),
  )(emb_flat, pos_rows, page_rows, qsel, seg, rsel,
    w1aw, w1wx, w1pos, w1page,
    b1.reshape(1, 8), g1.reshape(1, 8), be1.reshape(1, 8),
    W2.reshape(1, 8), b2.reshape(1, 1), g2.reshape(1, 1), be2.reshape(1, 1))
  return out


# speedup vs baseline: 3.4842x; 1.0014x over previous
import functools

import jax
import jax.numpy as jnp
import numpy as np
from jax import lax
from jax.experimental import pallas as pl
from jax.experimental.pallas import tpu as pltpu
from jax.experimental.pallas import tpu_sc as plsc

B = 16384
VOCAB = 1000000
EDIM = 16
NB = 7
NF = 2
NSLOT = NF * NB
D = NSLOT * EDIM

NC = 2
NS = 16
NW = NC * NS

CHUNK = 512
GROUP = 2048
EMB_PER_W = B * NSLOT // NW
POS_PER_W = B // NW

TBLK = 16384
NTB = VOCAB // TBLK
TAIL = VOCAB - NTB * TBLK


def _pack8(y):
  y3 = y.reshape(y.shape[0] // 8, 8, EDIM)
  return jnp.concatenate([y3[:, a, :] for a in range(8)], axis=1)


def _transpose_body(tt_ref, out_ref):
  x = tt_ref[...]
  out_ref[...] = _pack8(jnp.transpose(x))


def _sc_gather_body(emb_idx, pos_idx, page_idx,
                    emb_tab, pos_tab, page_tab,
                    emb_out, pos_out, page_out,
                    idx_v, pidx_v, buf0, buf1, g0, g1, s0, s1):
  wid = lax.axis_index("s") * NC + lax.axis_index("c")
  base = wid * EMB_PER_W
  pltpu.sync_copy(emb_idx.at[pl.ds(base, EMB_PER_W)], idx_v)

  bufs = (buf0, buf1)
  gsems = (g0, g1)
  ssems = (s0, s1)
  ngroups = EMB_PER_W // GROUP
  rem = EMB_PER_W - ngroups * GROUP
  store_handles = [None, None]
  for g in range(ngroups + 1):
    width = GROUP if g < ngroups else rem
    p = g % 2
    if store_handles[p] is not None:
      store_handles[p].wait()
    handles = []
    for j in range(width // CHUNK):
      off = g * GROUP + j * CHUNK
      handles.append(pltpu.async_copy(
          emb_tab.at[idx_v.at[pl.ds(off, CHUNK)]],
          bufs[p].at[pl.ds(j * CHUNK, CHUNK)], gsems[p]))
    for h in handles:
      h.wait()
    store_handles[p] = pltpu.async_copy(
        bufs[p].at[pl.ds(0, width)],
        emb_out.at[pl.ds(base + g * GROUP, width)], ssems[p])
  for h in store_handles:
    if h is not None:
      h.wait()

  pbase = wid * POS_PER_W
  for src_idx, tab, out, buf, gsem, ssem in (
      (pos_idx, pos_tab, pos_out, buf0, g0, s0),
      (page_idx, page_tab, page_out, buf1, g1, s1),
  ):
    pltpu.sync_copy(src_idx.at[pl.ds(pbase, POS_PER_W)], pidx_v)
    pltpu.async_copy(
        tab.at[pidx_v], buf.at[pl.ds(0, POS_PER_W)], gsem).wait()
    pltpu.async_copy(
        buf.at[pl.ds(0, POS_PER_W)], out.at[pl.ds(pbase, POS_PER_W)],
        ssem).wait()


@functools.lru_cache(maxsize=None)
def _make_sc_gather():
  return pl.kernel(
      _sc_gather_body,
      out_type=(
          jax.ShapeDtypeStruct((B * NSLOT, EDIM), jnp.float32),
          jax.ShapeDtypeStruct((B, EDIM), jnp.float32),
          jax.ShapeDtypeStruct((B, EDIM), jnp.float32),
      ),
      mesh=plsc.VectorSubcoreMesh(core_axis_name="c", subcore_axis_name="s"),
      compiler_params=pltpu.CompilerParams(use_tc_tiling_on_sc=False),
      scratch_types=[
          pltpu.VMEM((EMB_PER_W,), jnp.int32),
          pltpu.VMEM((POS_PER_W,), jnp.int32),
          pltpu.VMEM((GROUP, EDIM), jnp.float32),
          pltpu.VMEM((GROUP, EDIM), jnp.float32),
          pltpu.SemaphoreType.DMA,
          pltpu.SemaphoreType.DMA,
          pltpu.SemaphoreType.DMA,
          pltpu.SemaphoreType.DMA,
      ],
  )


@functools.lru_cache(maxsize=None)
def _selection_mats():
  i = np.arange(D)
  s = i // EDIM
  e = i % EDIM
  f = s // NB
  qsel = np.zeros((D, D), np.float32)
  qsel[(f * NB + 3) * EDIM + e, i] = 1.0
  seg = np.zeros((D, NSLOT), np.float32)
  seg[i, s] = 1.0
  rsel = seg.T.copy()
  asel = np.zeros((D, NF * EDIM), np.float32)
  asel[i, f * EDIM + e] = 1.0
  return (jnp.asarray(qsel), jnp.asarray(seg),
          jnp.asarray(rsel), jnp.asarray(asel))


def _tc_body(emb_ref, pos_ref, page_ref, qsel_ref, seg_ref, rsel_ref,
             w1aw_ref, w1wx_ref, w1pos_ref, w1page_ref, b1_ref, g1_ref,
             be1_ref, w2_ref, b2_ref, g2_ref, be2_ref, out_ref):
  x = emb_ref[...]
  qq = jax.lax.dot(x, qsel_ref[...], preferred_element_type=jnp.float32)
  prod = x * qq
  scores = jax.lax.dot(prod, seg_ref[...],
                       preferred_element_type=jnp.float32) * (1.0 / 4.0)

  def softmax7(sc):
    m = jnp.max(sc, axis=-1, keepdims=True)
    ex = jnp.exp(sc - m)
    return ex / jnp.sum(ex, axis=-1, keepdims=True)

  aw = jnp.concatenate(
      [softmax7(scores[:, :NB]), softmax7(scores[:, NB:2 * NB])], axis=1)

  w_rep = jax.lax.dot(aw, rsel_ref[...], preferred_element_type=jnp.float32)
  wx = x * w_rep
  h = (jax.lax.dot(aw, w1aw_ref[...], preferred_element_type=jnp.float32)
       + jax.lax.dot(wx, w1wx_ref[...], preferred_element_type=jnp.float32)
       + jax.lax.dot(pos_ref[...], w1pos_ref[...],
                     preferred_element_type=jnp.float32)
       + jax.lax.dot(page_ref[...], w1page_ref[...],
                     preferred_element_type=jnp.float32)
       + b1_ref[...])
  mu = jnp.mean(h, axis=-1, keepdims=True)
  var = jnp.mean((h - mu) ** 2, axis=-1, keepdims=True)
  h = g1_ref[...] * (h - mu) / jnp.sqrt(var + 1e-3) + be1_ref[...]
  h = jnp.maximum(h, 0.0)

  h2 = jnp.sum(h * w2_ref[...], axis=-1, keepdims=True) + b2_ref[...]
  mu2 = jnp.mean(h2, axis=-1, keepdims=True)
  var2 = jnp.mean((h2 - mu2) ** 2, axis=-1, keepdims=True)
  h2 = g2_ref[...] * (h2 - mu2) / jnp.sqrt(var2 + 1e-3) + be2_ref[...]
  out_ref[...] = jnp.maximum(h2, 0.0)


def _pack_table(neighbourhood_table):
  tt = neighbourhood_table.T
  packed = pl.pallas_call(
      _transpose_body,
      grid=(NTB,),
      in_specs=[pl.BlockSpec((EDIM, TBLK), lambda i: (0, i))],
      out_specs=pl.BlockSpec((TBLK // 8, 128), lambda i: (i, 0)),
      out_shape=jax.ShapeDtypeStruct((VOCAB // 8, 128), jnp.float32),
  )(tt)
  tail_packed = neighbourhood_table[NTB * TBLK:].reshape(TAIL // 8, 128)
  packed = lax.dynamic_update_slice(packed, tail_packed, (NTB * TBLK // 8, 0))
  return packed.reshape(VOCAB, EDIM)


def kernel(position, page, near_expo_seq_cate2, near_expo_seq_cate3,
           neighbourhood_table, position_table, page_table,
           W1, b1, g1, be1, W2, b2, g2, be2):
  ids = jnp.concatenate(
      [near_expo_seq_cate2, near_expo_seq_cate3], axis=1
  ).reshape(-1).astype(jnp.int32)
  pos_idx = position.astype(jnp.int32)
  page_idx = page.astype(jnp.int32)

  rowtab = _pack_table(neighbourhood_table)

  emb_rows, pos_rows, page_rows = _make_sc_gather()(
      ids, pos_idx, page_idx, rowtab, position_table, page_table)
  emb_flat = emb_rows.reshape(B, NSLOT * EDIM)

  qsel, seg, rsel, asel = _selection_mats()
  w1aw = W1[:NSLOT]
  w1wx = asel @ W1[NSLOT:NSLOT + NF * EDIM]
  w1pos = W1[NSLOT + NF * EDIM:NSLOT + NF * EDIM + EDIM]
  w1page = W1[NSLOT + NF * EDIM + EDIM:]

  blk = 2048
  grid = B // blk
  full = lambda i: (0, 0)
  row = lambda i: (i, 0)
  out = pl.pallas_call(
      _tc_body,
      grid=(grid,),
      in_specs=[
          pl.BlockSpec((blk, NSLOT * EDIM), row),
          pl.BlockSpec((blk, EDIM), row),
          pl.BlockSpec((blk, EDIM), row),
          pl.BlockSpec(qsel.shape, full),
          pl.BlockSpec(seg.shape, full),
          pl.BlockSpec(rsel.shape, full),
          pl.BlockSpec((NSLOT, 8), full),
          pl.BlockSpec((D, 8), full),
          pl.BlockSpec((EDIM, 8), full),
          pl.BlockSpec((EDIM, 8), full),
          pl.BlockSpec((1, 8), full),
          pl.BlockSpec((1, 8), full),
          pl.BlockSpec((1, 8), full),
          pl.BlockSpec((1, 8), full),
          pl.BlockSpec((1, 1), full),
          pl.BlockSpec((1, 1), full),
          pl.BlockSpec((1, 1), full),
      ],
      out_specs=pl.BlockSpec((blk, 1), row),
      out_shape=jax.ShapeDtypeStruct((B, 1), jnp.float32),
  )(emb_flat, pos_rows, page_rows, qsel, seg, rsel,
    w1aw, w1wx, w1pos, w1page,
    b1.reshape(1, 8), g1.reshape(1, 8), be1.reshape(1, 8),
    W2.reshape(1, 8), b2.reshape(1, 1), g2.reshape(1, 1), be2.reshape(1, 1))
  return out
